# baseline (speedup 1.0000x reference)
import jax
import jax.numpy as jnp
from jax import lax
from jax.experimental import pallas as pl
from jax.experimental.pallas import tpu as pltpu
from jax.experimental.pallas import tpu_sc as plsc

M, D, B = 100000, 64, 16384
L = 16
NG = D // L
NC = 2
NS = 16
GPC = NG // NC

UPD_ROWS = B // NS
SUB = 128
NSUB = UPD_ROWS // SUB
RPC = 8
SEG = M * L // NS
OUT_ROWS = M // NS
RCH = 125
NRCH = OUT_ROWS // RCH
ZCH = 2500
NZ = SEG // ZCH


def _body(x_hbm, y0_hbm, y1_hbm, z_hbm, out0_hbm, out1_hbm,
          acc, y_v, z_v, idx_v, val_v, zero_v, sbuf, xbuf, obuf,
          sem_y, sem_z, sem_sc, sem_a, sem_x, sem_o, sem_zero):
    c = lax.axis_index("c")
    s = lax.axis_index("s")
    iota = lax.iota(jnp.int32, L)
    seg0 = pl.multiple_of(s * SEG, 8)

    @pl.loop(0, ZCH // L)
    def _(i):
        zero_v[pl.ds(pl.multiple_of(i * L, L), L)] = jnp.zeros((L,), jnp.float32)

    for y_hbm, out_hbm in ((y0_hbm, out0_hbm), (y1_hbm, out1_hbm)):
        for gg in range(GPC):
            g = c * GPC + gg
            col0 = pl.multiple_of(g * L, L)

            def upd_load(sb, b, y_hbm=y_hbm):
                r0 = s * UPD_ROWS + sb * SUB
                pltpu.async_copy(
                    y_hbm.at[pl.ds(r0, SUB), pl.ds(col0, L)],
                    y_v.at[b], sem_y)
                pltpu.async_copy(
                    z_hbm.at[pl.ds(r0, SUB), pl.ds(col0, L)],
                    z_v.at[b], sem_z)

            def upd_load_wait(b, y_hbm=y_hbm):
                pltpu.make_async_copy(
                    y_hbm.at[pl.ds(0, SUB), pl.ds(col0, L)],
                    y_v.at[b], sem_y).wait()
                pltpu.make_async_copy(
                    z_hbm.at[pl.ds(0, SUB), pl.ds(col0, L)],
                    z_v.at[b], sem_z).wait()

            @pl.loop(0, NZ)
            def _(k):
                off = pl.multiple_of(seg0 + k * ZCH, 8)
                pltpu.async_copy(zero_v, acc.at[pl.ds(off, ZCH)], sem_zero)

            upd_load(0, 0)

            @pl.loop(0, NZ)
            def _(k):
                off = pl.multiple_of(seg0 + k * ZCH, 8)
                pltpu.make_async_copy(
                    zero_v, acc.at[pl.ds(off, ZCH)], sem_zero).wait()

            plsc.subcore_barrier()

            @pl.loop(0, NSUB)
            def _(sb):
                b = lax.rem(sb, 2)
                upd_load_wait(b)

                @pl.loop(0, SUB // RPC)
                def _(cc):
                    for u in range(RPC):
                        r = cc * RPC + u
                        off = pl.multiple_of((r % RPC) * L, L)
                        idx_v[cc, pl.ds(off, L)] = y_v[b, r, :] * L + iota
                        val_v[cc, pl.ds(off, L)] = z_v[b, r, :]

                @pl.loop(0, SUB // RPC)
                def _(cc):
                    pltpu.async_copy(val_v.at[cc],
                                     acc.at[idx_v.at[cc]],
                                     sem_sc, add=True).wait()

                @pl.when(sb < NSUB - 1)
                def _():
                    upd_load(sb + 1, 1 - b)

            plsc.subcore_barrier()

            def ro_load(k, b):
                row0 = s * OUT_ROWS + k * RCH
                aoff = pl.multiple_of(seg0 + k * RCH * L, 8)
                pltpu.async_copy(acc.at[pl.ds(aoff, RCH * L)],
                                 sbuf.at[b], sem_a)
                pltpu.async_copy(
                    x_hbm.at[pl.ds(row0, RCH), pl.ds(col0, L)],
                    xbuf.at[b], sem_x)

            def ro_load_wait(b):
                pltpu.make_async_copy(acc.at[pl.ds(0, RCH * L)],
                                      sbuf.at[b], sem_a).wait()
                pltpu.make_async_copy(
                    x_hbm.at[pl.ds(0, RCH), pl.ds(col0, L)],
                    xbuf.at[b], sem_x).wait()

            def ro_store(k, b, out_hbm=out_hbm):
                row0 = s * OUT_ROWS + k * RCH
                pltpu.async_copy(
                    obuf.at[b],
                    out_hbm.at[pl.ds(row0, RCH), pl.ds(col0, L)], sem_o)

            def ro_store_wait(b, out_hbm=out_hbm):
                pltpu.make_async_copy(
                    obuf.at[b],
                    out_hbm.at[pl.ds(0, RCH), pl.ds(col0, L)], sem_o).wait()

            ro_load(0, 0)

            @pl.loop(0, NRCH)
            def _(k):
                b = lax.rem(k, 2)

                @pl.when(k < NRCH - 1)
                def _():
                    ro_load(k + 1, 1 - b)

                ro_load_wait(b)

                @pl.when(k > 1)
                def _():
                    ro_store_wait(b)

                @pl.loop(0, RCH // 5)
                def _(rr):
                    for u in range(5):
                        r = rr * 5 + u
                        obuf[b, r, :] = (
                            sbuf[b, pl.ds(pl.multiple_of(r * L, L), L)]
                            + xbuf[b, r, :])

                ro_store(k, b)

            ro_store_wait((NRCH - 2) % 2)
            ro_store_wait((NRCH - 1) % 2)


_sc_call = pl.kernel(
    _body,
    out_type=(
        jax.ShapeDtypeStruct((M, D), jnp.float32),
        jax.ShapeDtypeStruct((M, D), jnp.float32),
    ),
    mesh=plsc.VectorSubcoreMesh(core_axis_name="c", subcore_axis_name="s"),
    compiler_params=pltpu.CompilerParams(use_tc_tiling_on_sc=False),
    scratch_types=[
        pltpu.VMEM_SHARED((M * L,), jnp.float32),
        pltpu.VMEM((2, SUB, L), jnp.int32),
        pltpu.VMEM((2, SUB, L), jnp.float32),
        pltpu.VMEM((SUB // RPC, RPC * L), jnp.int32),
        pltpu.VMEM((SUB // RPC, RPC * L), jnp.float32),
        pltpu.VMEM((ZCH,), jnp.float32),
        pltpu.VMEM((2, RCH * L), jnp.float32),
        pltpu.VMEM((2, RCH, L), jnp.float32),
        pltpu.VMEM((2, RCH, L), jnp.float32),
        pltpu.SemaphoreType.DMA,
        pltpu.SemaphoreType.DMA,
        pltpu.SemaphoreType.DMA,
        pltpu.SemaphoreType.DMA,
        pltpu.SemaphoreType.DMA,
        pltpu.SemaphoreType.DMA,
        pltpu.SemaphoreType.DMA,
    ],
)


def kernel(x, y0, y1, z):
    return _sc_call(x, y0, y1, z)
